# baseline (speedup 1.0000x reference)
import jax
import jax.numpy as jnp
from jax.experimental import pallas as pl
from jax.experimental.pallas import tpu as pltpu

B, D = 16384, 128
R = 1024
G = B // R


def _tc_body(mu, s2, mup, s2p, lab, acc, cnt):
    i = pl.program_id(0)

    @pl.when(i == 0)
    def _():
        acc[...] = jnp.zeros((8, D), jnp.float32)
        cnt[...] = jnp.zeros((8, D), jnp.float32)

    m = mu[...]
    v = s2[...]
    mp = mup[...]
    vp = s2p[...]
    d = v - vp
    dm = m - mp
    term = d - jnp.exp(d) - dm * dm * jnp.exp(-vp)
    mask = (lab[...] != 4).astype(jnp.float32).reshape(R, 1)
    acc[...] += jnp.sum((term * mask).reshape(R // 8, 8, D), axis=0)
    cnt[...] += jnp.sum(jnp.broadcast_to(mask, (R, D)).reshape(R // 8, 8, D),
                        axis=0)


@jax.jit
def _run(mu, sigma2, mu_pri, sigma2_pri, lab):
    blk = pl.BlockSpec((R, D), lambda i: (i, 0))
    out = pl.pallas_call(
        _tc_body,
        grid=(G,),
        in_specs=[blk, blk, blk, blk, pl.BlockSpec((R,), lambda i: (i,))],
        out_specs=[pl.BlockSpec((8, D), lambda i: (0, 0))] * 2,
        out_shape=[jax.ShapeDtypeStruct((8, D), jnp.float32)] * 2,
    )(mu, sigma2, mu_pri, sigma2_pri, lab)
    total = jnp.sum(out[0])
    n = jnp.sum(out[1]) / D
    loss = -0.5 * (total + n * D) / n
    return jnp.where(n > 0, loss, jnp.float32(0.0))


def kernel(mu, sigma2, mu_pri, sigma2_pri, style_label):
    return _run(mu, sigma2, mu_pri, sigma2_pri,
                style_label.astype(jnp.int32))


# speedup vs baseline: 1.2720x; 1.2720x over previous
import jax
import jax.numpy as jnp
from jax.experimental import pallas as pl
from jax.experimental.pallas import tpu as pltpu

B, D = 16384, 128
R = 4096
G = B // R


def _tc_body(mu, s2, mup, s2p, lab, acc, cnt):
    i = pl.program_id(0)

    @pl.when(i == 0)
    def _():
        acc[...] = jnp.zeros((8, D), jnp.float32)
        cnt[...] = jnp.zeros((8, D), jnp.float32)

    m = mu[...]
    v = s2[...]
    mp = mup[...]
    vp = s2p[...]
    d = v - vp
    dm = m - mp
    term = d - jnp.exp(d) - dm * dm * jnp.exp(-vp)
    mask = (lab[...] != 4).astype(jnp.float32).reshape(R, 1)
    acc[...] += jnp.sum((term * mask).reshape(R // 8, 8, D), axis=0)
    cnt[...] += jnp.sum(jnp.broadcast_to(mask, (R, D)).reshape(R // 8, 8, D),
                        axis=0)


@jax.jit
def _run(mu, sigma2, mu_pri, sigma2_pri, lab):
    blk = pl.BlockSpec((R, D), lambda i: (i, 0))
    out = pl.pallas_call(
        _tc_body,
        grid=(G,),
        in_specs=[blk, blk, blk, blk, pl.BlockSpec((R,), lambda i: (i,))],
        out_specs=[pl.BlockSpec((8, D), lambda i: (0, 0))] * 2,
        out_shape=[jax.ShapeDtypeStruct((8, D), jnp.float32)] * 2,
    )(mu, sigma2, mu_pri, sigma2_pri, lab)
    total = jnp.sum(out[0])
    n = jnp.sum(out[1]) / D
    loss = -0.5 * (total + n * D) / n
    return jnp.where(n > 0, loss, jnp.float32(0.0))


def kernel(mu, sigma2, mu_pri, sigma2_pri, style_label):
    return _run(mu, sigma2, mu_pri, sigma2_pri,
                style_label.astype(jnp.int32))
